# baseline (speedup 1.0000x reference)
import functools

import jax
import jax.numpy as jnp
from jax import lax
from jax.experimental import pallas as pl
from jax.experimental.pallas import tpu as pltpu
from jax.experimental.pallas import tpu_sc as plsc

N = 10000
E = 320000
D = 128
H = 128
NEG_SLOPE = 0.2
EP = E + N

NC, NS, L = 2, 16, 16
NW = NC * NS
CH = 64
RND = 162
PER = CH * RND
EPAD = NW * PER
NPAD = 10240
GRP = CH // L


def _proj_body(x_ref, w1a_ref, w1b_ref, b1_ref, wa1_ref, wa2_ref,
               u_ref, v_ref, s1_ref, s2_ref):
    xb = x_ref[...]
    u_ref[...] = jnp.dot(xb, w1a_ref[...],
                         preferred_element_type=jnp.float32) + b1_ref[...]
    v_ref[...] = jnp.dot(xb, w1b_ref[...], preferred_element_type=jnp.float32)
    s1_ref[...] = jnp.dot(xb, wa1_ref[...], preferred_element_type=jnp.float32)
    s2_ref[...] = jnp.dot(xb, wa2_ref[...], preferred_element_type=jnp.float32)


def _projections(xp, w1a, w1b, b1, wa1, wa2):
    nblk = NPAD // 128
    return pl.pallas_call(
        _proj_body,
        grid=(nblk,),
        in_specs=[
            pl.BlockSpec((128, D), lambda i: (i, 0)),
            pl.BlockSpec((D, H), lambda i: (0, 0)),
            pl.BlockSpec((D, H), lambda i: (0, 0)),
            pl.BlockSpec((1, H), lambda i: (0, 0)),
            pl.BlockSpec((D, 1), lambda i: (0, 0)),
            pl.BlockSpec((D, 1), lambda i: (0, 0)),
        ],
        out_specs=[
            pl.BlockSpec((128, H), lambda i: (i, 0)),
            pl.BlockSpec((128, H), lambda i: (i, 0)),
            pl.BlockSpec((128, 1), lambda i: (i, 0)),
            pl.BlockSpec((128, 1), lambda i: (i, 0)),
        ],
        out_shape=[
            jax.ShapeDtypeStruct((NPAD, H), jnp.float32),
            jax.ShapeDtypeStruct((NPAD, H), jnp.float32),
            jax.ShapeDtypeStruct((NPAD, 1), jnp.float32),
            jax.ShapeDtypeStruct((NPAD, 1), jnp.float32),
        ],
    )(xp, w1a, w1b, b1, wa1, wa2)


def _sc_body(ub_h, v_h, s1_h, s2_h, idx_h, wc_h,
             kvt_h, p1_h, p2_h, p3_h, seg_h,
             idx_v, ubuf, vbuf,
             p1_v, p2_v, p3_v,
             s1_v, s2_v, wc_v, kvt_v, seg_v, sem):
    wid = lax.axis_index("s") * NC + lax.axis_index("c")
    base = wid * PER

    pltpu.sync_copy(s1_h, s1_v)
    pltpu.sync_copy(s2_h, s2_v)
    pltpu.sync_copy(wc_h, wc_v)
    pltpu.sync_copy(kvt_h, kvt_v)

    zeros16 = jnp.zeros((L,), jnp.float32)

    def _zero(i, _):
        seg_v[pl.ds(i * L, L)] = zeros16
        return 0

    lax.fori_loop(0, NPAD // L, _zero, 0)

    def _round(r, _):
        off = base + r * CH
        pltpu.sync_copy(idx_h.at[wid * RND + r], idx_v)
        c1 = pltpu.async_copy(ub_h.at[idx_v.at[pl.ds(0, CH)]], ubuf, sem)
        c2 = pltpu.async_copy(v_h.at[idx_v.at[pl.ds(2 * CH, 3 * CH)]],
                              vbuf, sem)
        c1.wait()
        c2.wait()

        b2v = wc_v[H]

        for g in range(GRP):
            eids = jnp.arange(L, dtype=jnp.int32) + (g * L)

            def _feat(k, accs):
                a1, a2, a3 = accs
                kv = kvt_v[k]
                u = plsc.load_gather(ubuf, [eids, kv])
                vv = plsc.load_gather(vbuf, [eids, kv])
                w1n = plsc.load_gather(vbuf, [eids + CH, kv])
                w2n = plsc.load_gather(vbuf, [eids + 2 * CH, kv])
                z1 = u + vv
                z2 = u + w1n
                z3 = u + w2n
                l1 = jnp.where(z1 > 0, z1, z1 * NEG_SLOPE)
                l2 = jnp.where(z2 > 0, z2, z2 * NEG_SLOPE)
                l3 = jnp.where(z3 > 0, z3, z3 * NEG_SLOPE)
                wk = wc_v[k]
                return (a1 + l1 * wk, a2 + l2 * wk, a3 + l3 * wk)

            a1, a2, a3 = lax.fori_loop(0, H, _feat,
                                       (zeros16, zeros16, zeros16))
            p1 = a1 + b2v
            p2 = a2 + b2v
            p3 = a3 + b2v

            rg = idx_v[pl.ds(g * L, L)]
            cg = idx_v[pl.ds(2 * CH + g * L, L)]
            sa = plsc.load_gather(s1_v, [rg])
            sb = plsc.load_gather(s2_v, [cg])
            att = 1.0 / (1.0 + jnp.exp(-(sa + sb)))

            rs = idx_v[pl.ds(CH + g * L, L)]
            plsc.addupdate_scatter(seg_v, [rs], p1 * att)

            p1_v[pl.ds(g * L, L)] = p1
            p2_v[pl.ds(g * L, L)] = p2
            p3_v[pl.ds(g * L, L)] = p3

        pltpu.sync_copy(p1_v, p1_h.at[pl.ds(off, CH)])
        pltpu.sync_copy(p2_v, p2_h.at[pl.ds(off, CH)])
        pltpu.sync_copy(p3_v, p3_h.at[pl.ds(off, CH)])
        return 0

    lax.fori_loop(0, RND, _round, 0)
    pltpu.sync_copy(seg_v, seg_h.at[wid])


_sc_edge_sweep = functools.partial(
    pl.kernel,
    out_type=[
        jax.ShapeDtypeStruct((EPAD,), jnp.float32),
        jax.ShapeDtypeStruct((EPAD,), jnp.float32),
        jax.ShapeDtypeStruct((EPAD,), jnp.float32),
        jax.ShapeDtypeStruct((NW, NPAD), jnp.float32),
    ],
    mesh=plsc.VectorSubcoreMesh(core_axis_name="c", subcore_axis_name="s"),
    compiler_params=pltpu.CompilerParams(use_tc_tiling_on_sc=False,
                                         needs_layout_passes=False),
    scratch_types=[
        pltpu.VMEM((5 * CH,), jnp.int32),
        pltpu.VMEM((CH, H), jnp.float32),
        pltpu.VMEM((3 * CH, H), jnp.float32),
        pltpu.VMEM((CH,), jnp.float32),
        pltpu.VMEM((CH,), jnp.float32),
        pltpu.VMEM((CH,), jnp.float32),
        pltpu.VMEM((NPAD,), jnp.float32),
        pltpu.VMEM((NPAD,), jnp.float32),
        pltpu.VMEM((H + 1, L), jnp.float32),
        pltpu.VMEM((H, L), jnp.int32),
        pltpu.VMEM((NPAD,), jnp.float32),
        pltpu.SemaphoreType.DMA,
    ],
)(_sc_body)


def _red_body(p1_ref, p2_ref, p3_ref, seg_ref, scores_ref, loss_ref):
    rows = EPAD // 128
    idx = (lax.broadcasted_iota(jnp.int32, (rows, 128), 0) * 128
           + lax.broadcasted_iota(jnp.int32, (rows, 128), 1))
    valid = idx < EP

    def softplus(x):
        return jnp.maximum(x, 0.0) + jnp.log(1.0 + jnp.exp(-jnp.abs(x)))

    s_pos = jnp.sum(jnp.where(valid, softplus(-p1_ref[...]), 0.0))
    s_neg = jnp.sum(jnp.where(valid, softplus(p2_ref[...]), 0.0)) \
        + jnp.sum(jnp.where(valid, softplus(p3_ref[...]), 0.0))
    loss = s_pos / EP + s_neg / (2 * EP)
    loss_ref[...] = jnp.full((8, 128), loss)

    seg = jnp.sum(seg_ref[...], axis=0)
    scores_ref[...] = 1.0 / (1.0 + jnp.exp(-seg))


def _reduce(p1, p2, p3, segp):
    rows = EPAD // 128
    return pl.pallas_call(
        _red_body,
        in_specs=[
            pl.BlockSpec((rows, 128), lambda: (0, 0)),
            pl.BlockSpec((rows, 128), lambda: (0, 0)),
            pl.BlockSpec((rows, 128), lambda: (0, 0)),
            pl.BlockSpec((NW, NPAD), lambda: (0, 0)),
        ],
        out_specs=[
            pl.BlockSpec((NPAD,), lambda: (0,)),
            pl.BlockSpec((8, 128), lambda: (0, 0)),
        ],
        out_shape=[
            jax.ShapeDtypeStruct((NPAD,), jnp.float32),
            jax.ShapeDtypeStruct((8, 128), jnp.float32),
        ],
    )(p1, p2, p3, segp)


def kernel(x, edge_index, att_weight, W1, b1, W2, b2):
    loops = jnp.arange(N, dtype=edge_index.dtype)
    row = jnp.concatenate([edge_index[0], loops])
    col = jnp.concatenate([edge_index[1], loops])

    neg = []
    for i in range(2):
        nk = jax.random.fold_in(jax.random.key(42), i)
        neg.append(jax.random.randint(nk, row.shape, 0, N, dtype=row.dtype))

    pad = EPAD - EP

    def shard(a, padval):
        return jnp.concatenate([a, jnp.full((pad,), padval, a.dtype)]) \
            .reshape(NW, RND, CH)

    idx_h = jnp.stack(
        [shard(row, 0), shard(row, N), shard(col, 0),
         shard(neg[0], 0), shard(neg[1], 0)],
        axis=2).reshape(NW * RND, 5 * CH)

    xp = jnp.pad(x, ((0, NPAD - N), (0, 0)))
    w1a = W1[:D]
    w1b = W1[D:]
    wa1 = att_weight[0, :D].reshape(D, 1)
    wa2 = att_weight[0, D:].reshape(D, 1)

    ub, v, s1, s2 = _projections(xp, w1a, w1b, b1.reshape(1, H), wa1, wa2)

    ks = jnp.arange(H, dtype=jnp.int32).reshape(H, 1)
    lanes16 = jnp.arange(L, dtype=jnp.int32).reshape(1, L)
    kvtab = (ks & ~(L - 1)) + ((ks + lanes16) & (L - 1))
    wc = jnp.concatenate([W2[:, 0][kvtab],
                          jnp.tile(b2.reshape(1, 1), (1, L))], axis=0)

    p1, p2, p3, segp = _sc_edge_sweep(
        ub, v, s1.reshape(NPAD), s2.reshape(NPAD), idx_h, wc, kvtab)

    rows = EPAD // 128
    scores_full, loss_buf = _reduce(
        p1.reshape(rows, 128), p2.reshape(rows, 128), p3.reshape(rows, 128),
        segp)

    scores = scores_full[:N].reshape(N, 1)
    loss = loss_buf[0, 0]
    return (scores, loss)


# speedup vs baseline: 1.1551x; 1.1551x over previous
import functools

import jax
import jax.numpy as jnp
from jax import lax
from jax.experimental import pallas as pl
from jax.experimental.pallas import tpu as pltpu
from jax.experimental.pallas import tpu_sc as plsc

N = 10000
E = 320000
D = 128
H = 128
NEG_SLOPE = 0.2
EP = E + N

NC, NS, L = 2, 16, 16
NW = NC * NS
CH = 64
RND = 162
RNDP = RND + 2
PER = CH * RND
EPAD = NW * PER
NPAD = 10240
GRP = CH // L


def _proj_body(x_ref, w1a_ref, w1b_ref, b1_ref, wa1_ref, wa2_ref,
               u_ref, v_ref, s1_ref, s2_ref):
    xb = x_ref[...]
    u_ref[...] = jnp.dot(xb, w1a_ref[...],
                         preferred_element_type=jnp.float32) + b1_ref[...]
    v_ref[...] = jnp.dot(xb, w1b_ref[...], preferred_element_type=jnp.float32)
    s1_ref[...] = jnp.dot(xb, wa1_ref[...], preferred_element_type=jnp.float32)
    s2_ref[...] = jnp.dot(xb, wa2_ref[...], preferred_element_type=jnp.float32)


def _projections(xp, w1a, w1b, b1, wa1, wa2):
    nblk = NPAD // 128
    return pl.pallas_call(
        _proj_body,
        grid=(nblk,),
        in_specs=[
            pl.BlockSpec((128, D), lambda i: (i, 0)),
            pl.BlockSpec((D, H), lambda i: (0, 0)),
            pl.BlockSpec((D, H), lambda i: (0, 0)),
            pl.BlockSpec((1, H), lambda i: (0, 0)),
            pl.BlockSpec((D, 1), lambda i: (0, 0)),
            pl.BlockSpec((D, 1), lambda i: (0, 0)),
        ],
        out_specs=[
            pl.BlockSpec((128, H), lambda i: (i, 0)),
            pl.BlockSpec((128, H), lambda i: (i, 0)),
            pl.BlockSpec((128, 1), lambda i: (i, 0)),
            pl.BlockSpec((128, 1), lambda i: (i, 0)),
        ],
        out_shape=[
            jax.ShapeDtypeStruct((NPAD, H), jnp.float32),
            jax.ShapeDtypeStruct((NPAD, H), jnp.float32),
            jax.ShapeDtypeStruct((NPAD, 1), jnp.float32),
            jax.ShapeDtypeStruct((NPAD, 1), jnp.float32),
        ],
    )(xp, w1a, w1b, b1, wa1, wa2)


def _sc_body(ub_h, v_h, s1_h, s2_h, gidx_h, wc_h, kvt_h,
             pp_h, seg_h,
             gidxA, gidxB, valk, uA, uB, vA, vB, pstA, pstB,
             s1_v, s2_v, wc_v, kvt_v, seg_v,
             gsem, isem, psem):
    wid = lax.axis_index("s") * NC + lax.axis_index("c")
    irow = wid * RNDP

    pltpu.sync_copy(s1_h, s1_v)
    pltpu.sync_copy(s2_h, s2_v)
    pltpu.sync_copy(wc_h, wc_v)
    pltpu.sync_copy(kvt_h, kvt_v)

    zeros16 = jnp.zeros((L,), jnp.float32)

    def _zero(i, _):
        seg_v[pl.ds(i * L, L)] = zeros16
        return 0

    lax.fori_loop(0, NPAD // L, _zero, 0)

    def launch_gathers(gidx_v, ubuf, vbuf):
        pltpu.async_copy(ub_h.at[gidx_v.at[pl.ds(0, CH)]], ubuf, gsem)
        pltpu.async_copy(v_h.at[gidx_v.at[pl.ds(CH, 3 * CH)]], vbuf, gsem)

    def drain_gathers(ubuf, vbuf):
        pltpu.make_async_copy(ub_h.at[pl.ds(0, CH)], ubuf, gsem).wait()
        pltpu.make_async_copy(v_h.at[pl.ds(0, 3 * CH)], vbuf, gsem).wait()

    pltpu.sync_copy(gidx_h.at[irow], gidxA)
    pltpu.async_copy(gidx_h.at[irow + 1], gidxB, isem)
    launch_gathers(gidxA, uA, vA)
    pltpu.async_copy(pstA, pp_h.at[irow + RND], psem)

    def phase(r, gidx_p, u_p, v_p, pst_p, gidx_q, u_q, v_q):
        drain_gathers(u_p, v_p)
        for t in range(2 * CH // L):
            valk[pl.ds(t * L, L)] = gidx_p[pl.ds(t * L, L)]
        pltpu.async_copy(gidx_h.at[irow + r + 2], gidx_p, isem)
        pltpu.make_async_copy(gidx_h.at[irow], gidx_q, isem).wait()
        launch_gathers(gidx_q, u_q, v_q)
        pltpu.make_async_copy(pst_p, pp_h.at[irow + RND], psem).wait()

        b2v = wc_v[H]
        ebase = wid * PER + r * CH

        for g in range(GRP):
            eids = jnp.arange(L, dtype=jnp.int32) + (g * L)

            def _feat(k, accs):
                a1, a2, a3 = accs
                kv = kvt_v[k]
                u = plsc.load_gather(u_p, [eids, kv])
                vv = plsc.load_gather(v_p, [eids, kv])
                w1n = plsc.load_gather(v_p, [eids + CH, kv])
                w2n = plsc.load_gather(v_p, [eids + 2 * CH, kv])
                z1 = u + vv
                z2 = u + w1n
                z3 = u + w2n
                l1 = jnp.where(z1 > 0, z1, z1 * NEG_SLOPE)
                l2 = jnp.where(z2 > 0, z2, z2 * NEG_SLOPE)
                l3 = jnp.where(z3 > 0, z3, z3 * NEG_SLOPE)
                wk = wc_v[k]
                return (a1 + l1 * wk, a2 + l2 * wk, a3 + l3 * wk)

            a1, a2, a3 = lax.fori_loop(0, H, _feat,
                                       (zeros16, zeros16, zeros16))
            p1 = a1 + b2v
            p2 = a2 + b2v
            p3 = a3 + b2v

            rs = valk[pl.ds(g * L, L)]
            cg = valk[pl.ds(CH + g * L, L)]
            sa = plsc.load_gather(s1_v, [rs])
            sb = plsc.load_gather(s2_v, [cg])
            att = 1.0 / (1.0 + jnp.exp(-(sa + sb)))

            valid = (eids + ebase) < EP
            plsc.addupdate_scatter(seg_v, [rs], p1 * att, mask=valid)

            pst_p[pl.ds(g * L, L)] = p1
            pst_p[pl.ds(CH + g * L, L)] = p2
            pst_p[pl.ds(2 * CH + g * L, L)] = p3

        pltpu.async_copy(pst_p, pp_h.at[irow + r], psem)

    def _iter(i, _):
        phase(2 * i, gidxA, uA, vA, pstA, gidxB, uB, vB)
        phase(2 * i + 1, gidxB, uB, vB, pstB, gidxA, uA, vA)
        return 0

    lax.fori_loop(0, RND // 2, _iter, 0)

    drain_gathers(uA, vA)
    pltpu.make_async_copy(gidx_h.at[irow], gidxA, isem).wait()
    pltpu.make_async_copy(pstA, pp_h.at[irow + RND], psem).wait()
    pltpu.sync_copy(seg_v, seg_h.at[wid])


_sc_edge_sweep = functools.partial(
    pl.kernel,
    out_type=[
        jax.ShapeDtypeStruct((NW * RNDP, 3 * CH), jnp.float32),
        jax.ShapeDtypeStruct((NW, NPAD), jnp.float32),
    ],
    mesh=plsc.VectorSubcoreMesh(core_axis_name="c", subcore_axis_name="s"),
    compiler_params=pltpu.CompilerParams(use_tc_tiling_on_sc=False,
                                         needs_layout_passes=False),
    scratch_types=[
        pltpu.VMEM((4 * CH,), jnp.int32),
        pltpu.VMEM((4 * CH,), jnp.int32),
        pltpu.VMEM((2 * CH,), jnp.int32),
        pltpu.VMEM((CH, H), jnp.float32),
        pltpu.VMEM((CH, H), jnp.float32),
        pltpu.VMEM((3 * CH, H), jnp.float32),
        pltpu.VMEM((3 * CH, H), jnp.float32),
        pltpu.VMEM((3 * CH,), jnp.float32),
        pltpu.VMEM((3 * CH,), jnp.float32),
        pltpu.VMEM((NPAD,), jnp.float32),
        pltpu.VMEM((NPAD,), jnp.float32),
        pltpu.VMEM((H + 1, L), jnp.float32),
        pltpu.VMEM((H, L), jnp.int32),
        pltpu.VMEM((NPAD,), jnp.float32),
        pltpu.SemaphoreType.DMA,
        pltpu.SemaphoreType.DMA,
        pltpu.SemaphoreType.DMA,
    ],
)(_sc_body)


def _red_body(p1_ref, p2_ref, p3_ref, seg_ref, scores_ref, loss_ref):
    rows = EPAD // 128
    idx = (lax.broadcasted_iota(jnp.int32, (rows, 128), 0) * 128
           + lax.broadcasted_iota(jnp.int32, (rows, 128), 1))
    valid = idx < EP

    def softplus(x):
        return jnp.maximum(x, 0.0) + jnp.log(1.0 + jnp.exp(-jnp.abs(x)))

    s_pos = jnp.sum(jnp.where(valid, softplus(-p1_ref[...]), 0.0))
    s_neg = jnp.sum(jnp.where(valid, softplus(p2_ref[...]), 0.0)) \
        + jnp.sum(jnp.where(valid, softplus(p3_ref[...]), 0.0))
    loss = s_pos / EP + s_neg / (2 * EP)
    loss_ref[...] = jnp.full((8, 128), loss)

    seg = jnp.sum(seg_ref[...], axis=0)
    scores_ref[...] = 1.0 / (1.0 + jnp.exp(-seg))


def _reduce(p1, p2, p3, segp):
    rows = EPAD // 128
    return pl.pallas_call(
        _red_body,
        in_specs=[
            pl.BlockSpec((rows, 128), lambda: (0, 0)),
            pl.BlockSpec((rows, 128), lambda: (0, 0)),
            pl.BlockSpec((rows, 128), lambda: (0, 0)),
            pl.BlockSpec((NW, NPAD), lambda: (0, 0)),
        ],
        out_specs=[
            pl.BlockSpec((NPAD,), lambda: (0,)),
            pl.BlockSpec((8, 128), lambda: (0, 0)),
        ],
        out_shape=[
            jax.ShapeDtypeStruct((NPAD,), jnp.float32),
            jax.ShapeDtypeStruct((8, 128), jnp.float32),
        ],
    )(p1, p2, p3, segp)


def kernel(x, edge_index, att_weight, W1, b1, W2, b2):
    loops = jnp.arange(N, dtype=edge_index.dtype)
    row = jnp.concatenate([edge_index[0], loops])
    col = jnp.concatenate([edge_index[1], loops])

    neg = []
    for i in range(2):
        nk = jax.random.fold_in(jax.random.key(42), i)
        neg.append(jax.random.randint(nk, row.shape, 0, N, dtype=row.dtype))

    pad = EPAD - EP

    def shard(a):
        return jnp.concatenate([a, jnp.zeros((pad,), a.dtype)]) \
            .reshape(NW, RND, CH)

    gidx_h = jnp.pad(
        jnp.stack([shard(row), shard(col), shard(neg[0]), shard(neg[1])],
                  axis=2),
        ((0, 0), (0, RNDP - RND), (0, 0), (0, 0))).reshape(NW * RNDP, 4 * CH)

    xp = jnp.pad(x, ((0, NPAD - N), (0, 0)))
    w1a = W1[:D]
    w1b = W1[D:]
    wa1 = att_weight[0, :D].reshape(D, 1)
    wa2 = att_weight[0, D:].reshape(D, 1)

    ub, v, s1, s2 = _projections(xp, w1a, w1b, b1.reshape(1, H), wa1, wa2)

    ks = jnp.arange(H, dtype=jnp.int32).reshape(H, 1)
    lanes16 = jnp.arange(L, dtype=jnp.int32).reshape(1, L)
    kvtab = (ks & ~(L - 1)) + ((ks + lanes16) & (L - 1))
    wc = jnp.concatenate([W2[:, 0][kvtab],
                          jnp.tile(b2.reshape(1, 1), (1, L))], axis=0)

    pp, segp = _sc_edge_sweep(
        ub, v, s1.reshape(NPAD), s2.reshape(NPAD), gidx_h, wc, kvtab)

    ppr = pp.reshape(NW, RNDP, 3, CH)[:, :RND]
    rows = EPAD // 128
    scores_full, loss_buf = _reduce(
        ppr[:, :, 0].reshape(rows, 128), ppr[:, :, 1].reshape(rows, 128),
        ppr[:, :, 2].reshape(rows, 128), segp)

    scores = scores_full[:N].reshape(N, 1)
    loss = loss_buf[0, 0]
    return (scores, loss)


# speedup vs baseline: 1.1562x; 1.0009x over previous
import functools

import jax
import jax.numpy as jnp
from jax import lax
from jax.experimental import pallas as pl
from jax.experimental.pallas import tpu as pltpu
from jax.experimental.pallas import tpu_sc as plsc

N = 10000
E = 320000
D = 128
H = 128
NEG_SLOPE = 0.2
EP = E + N

NC, NS, L = 2, 16, 16
NW = NC * NS
CH = 64
RND = 162
RNDP = RND + 2
PER = CH * RND
EPAD = NW * PER
NPAD = 10240
GRP = CH // L


def _proj_body(x_ref, w1a_ref, w1b_ref, b1_ref, wa1_ref, wa2_ref,
               u_ref, v_ref, s1_ref, s2_ref):
    xb = x_ref[...]
    u_ref[...] = jnp.dot(xb, w1a_ref[...],
                         preferred_element_type=jnp.float32) + b1_ref[...]
    v_ref[...] = jnp.dot(xb, w1b_ref[...], preferred_element_type=jnp.float32)
    s1_ref[...] = jnp.dot(xb, wa1_ref[...], preferred_element_type=jnp.float32)
    s2_ref[...] = jnp.dot(xb, wa2_ref[...], preferred_element_type=jnp.float32)


def _projections(xp, w1a, w1b, b1, wa1, wa2):
    nblk = NPAD // 128
    return pl.pallas_call(
        _proj_body,
        grid=(nblk,),
        in_specs=[
            pl.BlockSpec((128, D), lambda i: (i, 0)),
            pl.BlockSpec((D, H), lambda i: (0, 0)),
            pl.BlockSpec((D, H), lambda i: (0, 0)),
            pl.BlockSpec((1, H), lambda i: (0, 0)),
            pl.BlockSpec((D, 1), lambda i: (0, 0)),
            pl.BlockSpec((D, 1), lambda i: (0, 0)),
        ],
        out_specs=[
            pl.BlockSpec((128, H), lambda i: (i, 0)),
            pl.BlockSpec((128, H), lambda i: (i, 0)),
            pl.BlockSpec((128, 1), lambda i: (i, 0)),
            pl.BlockSpec((128, 1), lambda i: (i, 0)),
        ],
        out_shape=[
            jax.ShapeDtypeStruct((NPAD, H), jnp.float32),
            jax.ShapeDtypeStruct((NPAD, H), jnp.float32),
            jax.ShapeDtypeStruct((NPAD, 1), jnp.float32),
            jax.ShapeDtypeStruct((NPAD, 1), jnp.float32),
        ],
    )(xp, w1a, w1b, b1, wa1, wa2)


def _sc_body(ub_h, v_h, s1_h, s2_h, gidx_h, wc_h, kvt_h,
             pp_h, seg_h,
             gidxA, gidxB, valk, uA, uB, vA, vB, pstA, pstB,
             s1_v, s2_v, wc_v, kvt_v, seg_v,
             gsem, isem, psem):
    wid = lax.axis_index("s") * NC + lax.axis_index("c")
    irow = wid * RNDP

    pltpu.sync_copy(s1_h, s1_v)
    pltpu.sync_copy(s2_h, s2_v)
    pltpu.sync_copy(wc_h, wc_v)
    pltpu.sync_copy(kvt_h, kvt_v)

    zeros16 = jnp.zeros((L,), jnp.float32)

    def _zero(i, _):
        seg_v[pl.ds(i * L, L)] = zeros16
        return 0

    lax.fori_loop(0, NPAD // L, _zero, 0)

    def launch_gathers(gidx_v, ubuf, vbuf):
        pltpu.async_copy(ub_h.at[gidx_v.at[pl.ds(0, CH)]], ubuf, gsem)
        pltpu.async_copy(v_h.at[gidx_v.at[pl.ds(CH, 3 * CH)]], vbuf, gsem)

    def drain_gathers(ubuf, vbuf):
        pltpu.make_async_copy(ub_h.at[pl.ds(0, CH)], ubuf, gsem).wait()
        pltpu.make_async_copy(v_h.at[pl.ds(0, 3 * CH)], vbuf, gsem).wait()

    pltpu.sync_copy(gidx_h.at[irow], gidxA)
    pltpu.async_copy(gidx_h.at[irow + 1], gidxB, isem)
    launch_gathers(gidxA, uA, vA)
    pltpu.async_copy(pstA, pp_h.at[irow + RND], psem)

    def phase(r, gidx_p, u_p, v_p, pst_p, gidx_q, u_q, v_q):
        drain_gathers(u_p, v_p)
        for t in range(2 * CH // L):
            valk[pl.ds(t * L, L)] = gidx_p[pl.ds(t * L, L)]
        pltpu.async_copy(gidx_h.at[irow + r + 2], gidx_p, isem)
        pltpu.make_async_copy(gidx_h.at[irow], gidx_q, isem).wait()
        launch_gathers(gidx_q, u_q, v_q)
        pltpu.make_async_copy(pst_p, pp_h.at[irow + RND], psem).wait()

        b2v = wc_v[H]
        ebase = wid * PER + r * CH

        for g in range(GRP):
            eids = jnp.arange(L, dtype=jnp.int32) + (g * L)

            def _feat(k, accs):
                a1, a2, a3 = accs
                kv = kvt_v[k]
                u = plsc.load_gather(u_p, [eids, kv])
                vv = plsc.load_gather(v_p, [eids, kv])
                w1n = plsc.load_gather(v_p, [eids + CH, kv])
                w2n = plsc.load_gather(v_p, [eids + 2 * CH, kv])
                z1 = u + vv
                z2 = u + w1n
                z3 = u + w2n
                l1 = jnp.where(z1 > 0, z1, z1 * NEG_SLOPE)
                l2 = jnp.where(z2 > 0, z2, z2 * NEG_SLOPE)
                l3 = jnp.where(z3 > 0, z3, z3 * NEG_SLOPE)
                wk = wc_v[k]
                return (a1 + l1 * wk, a2 + l2 * wk, a3 + l3 * wk)

            a1, a2, a3 = lax.fori_loop(0, H, _feat,
                                       (zeros16, zeros16, zeros16))
            p1 = a1 + b2v
            p2 = a2 + b2v
            p3 = a3 + b2v

            rs = valk[pl.ds(g * L, L)]
            cg = valk[pl.ds(CH + g * L, L)]
            sa = plsc.load_gather(s1_v, [rs])
            sb = plsc.load_gather(s2_v, [cg])
            att = 1.0 / (1.0 + jnp.exp(-(sa + sb)))

            valid = (eids + ebase) < EP
            plsc.addupdate_scatter(seg_v, [rs], p1 * att, mask=valid)

            pst_p[pl.ds(g * L, L)] = p1
            pst_p[pl.ds(CH + g * L, L)] = p2
            pst_p[pl.ds(2 * CH + g * L, L)] = p3

        pltpu.async_copy(pst_p, pp_h.at[irow + r], psem)

    def _iter(i, _):
        phase(2 * i, gidxA, uA, vA, pstA, gidxB, uB, vB)
        phase(2 * i + 1, gidxB, uB, vB, pstB, gidxA, uA, vA)
        return 0

    lax.fori_loop(0, RND // 2, _iter, 0)

    drain_gathers(uA, vA)
    pltpu.make_async_copy(gidx_h.at[irow], gidxA, isem).wait()
    pltpu.make_async_copy(pstA, pp_h.at[irow + RND], psem).wait()
    pltpu.sync_copy(seg_v, seg_h.at[wid])


_sc_edge_sweep = functools.partial(
    pl.kernel,
    out_type=[
        jax.ShapeDtypeStruct((NW * RNDP, 3 * CH), jnp.float32),
        jax.ShapeDtypeStruct((NW, NPAD), jnp.float32),
    ],
    mesh=plsc.VectorSubcoreMesh(core_axis_name="c", subcore_axis_name="s"),
    compiler_params=pltpu.CompilerParams(use_tc_tiling_on_sc=False,
                                         needs_layout_passes=False),
    scratch_types=[
        pltpu.VMEM((4 * CH,), jnp.int32),
        pltpu.VMEM((4 * CH,), jnp.int32),
        pltpu.VMEM((2 * CH,), jnp.int32),
        pltpu.VMEM((CH, H), jnp.float32),
        pltpu.VMEM((CH, H), jnp.float32),
        pltpu.VMEM((3 * CH, H), jnp.float32),
        pltpu.VMEM((3 * CH, H), jnp.float32),
        pltpu.VMEM((3 * CH,), jnp.float32),
        pltpu.VMEM((3 * CH,), jnp.float32),
        pltpu.VMEM((NPAD,), jnp.float32),
        pltpu.VMEM((NPAD,), jnp.float32),
        pltpu.VMEM((H + 1, L), jnp.float32),
        pltpu.VMEM((H, L), jnp.int32),
        pltpu.VMEM((NPAD,), jnp.float32),
        pltpu.SemaphoreType.DMA,
        pltpu.SemaphoreType.DMA,
        pltpu.SemaphoreType.DMA,
    ],
)(_sc_body)


def _red_body(p1_ref, p2_ref, p3_ref, seg_ref, scores_ref, loss_ref):
    rows = EPAD // 128
    idx = (lax.broadcasted_iota(jnp.int32, (rows, 128), 0) * 128
           + lax.broadcasted_iota(jnp.int32, (rows, 128), 1))
    valid = idx < EP

    def softplus(x):
        return jnp.maximum(x, 0.0) + jnp.log(1.0 + jnp.exp(-jnp.abs(x)))

    s_pos = jnp.sum(jnp.where(valid, softplus(-p1_ref[...]), 0.0))
    s_neg = jnp.sum(jnp.where(valid, softplus(p2_ref[...]), 0.0)) \
        + jnp.sum(jnp.where(valid, softplus(p3_ref[...]), 0.0))
    loss = s_pos / EP + s_neg / (2 * EP)
    loss_ref[...] = jnp.full((8, 128), loss)

    seg = jnp.sum(seg_ref[...], axis=0)
    scores_ref[...] = 1.0 / (1.0 + jnp.exp(-seg))


def _reduce(p1, p2, p3, segp):
    rows = EPAD // 128
    return pl.pallas_call(
        _red_body,
        in_specs=[
            pl.BlockSpec((rows, 128), lambda: (0, 0)),
            pl.BlockSpec((rows, 128), lambda: (0, 0)),
            pl.BlockSpec((rows, 128), lambda: (0, 0)),
            pl.BlockSpec((NW, NPAD), lambda: (0, 0)),
        ],
        out_specs=[
            pl.BlockSpec((NPAD,), lambda: (0,)),
            pl.BlockSpec((8, 128), lambda: (0, 0)),
        ],
        out_shape=[
            jax.ShapeDtypeStruct((NPAD,), jnp.float32),
            jax.ShapeDtypeStruct((8, 128), jnp.float32),
        ],
    )(p1, p2, p3, segp)


def kernel(x, edge_index, att_weight, W1, b1, W2, b2):
    loops = jnp.arange(N, dtype=edge_index.dtype)
    row = jnp.concatenate([edge_index[0], loops])
    col = jnp.concatenate([edge_index[1], loops])

    neg = []
    for i in range(2):
        nk = jax.random.fold_in(jax.random.key(42), i)
        neg.append(jax.random.randint(nk, row.shape, 0, N, dtype=row.dtype))

    pad = EPAD - EP

    def shard(a):
        return jnp.concatenate([a, jnp.zeros((pad,), a.dtype)]) \
            .reshape(NW, RND, CH)

    gidx_h = jnp.pad(
        jnp.stack([shard(row), shard(col), shard(neg[0]), shard(neg[1])],
                  axis=2),
        ((0, 0), (0, RNDP - RND), (0, 0), (0, 0))).reshape(NW * RNDP, 4 * CH)

    xp = jnp.pad(x, ((0, NPAD - N), (0, 0)))
    w1a = W1[:D]
    w1b = W1[D:]
    wa1 = att_weight[0, :D].reshape(D, 1)
    wa2 = att_weight[0, D:].reshape(D, 1)

    ub, v, s1, s2 = _projections(xp, w1a, w1b, b1.reshape(1, H), wa1, wa2)

    ks = jnp.arange(H, dtype=jnp.int32).reshape(H, 1)
    lanes16 = jnp.arange(L, dtype=jnp.int32).reshape(1, L)
    kvtab = (ks + 9 * lanes16) & (H - 1)
    wc = jnp.concatenate([W2[:, 0][kvtab],
                          jnp.tile(b2.reshape(1, 1), (1, L))], axis=0)

    pp, segp = _sc_edge_sweep(
        ub, v, s1.reshape(NPAD), s2.reshape(NPAD), gidx_h, wc, kvtab)

    ppr = pp.reshape(NW, RNDP, 3, CH)[:, :RND]
    rows = EPAD // 128
    scores_full, loss_buf = _reduce(
        ppr[:, :, 0].reshape(rows, 128), ppr[:, :, 1].reshape(rows, 128),
        ppr[:, :, 2].reshape(rows, 128), segp)

    scores = scores_full[:N].reshape(N, 1)
    loss = loss_buf[0, 0]
    return (scores, loss)
